# baseline (speedup 1.0000x reference)
import functools

import jax
import jax.numpy as jnp
from jax import lax
from jax.experimental import pallas as pl
from jax.experimental.pallas import tpu as pltpu
from jax.experimental.pallas import tpu_sc as plsc

B = 4096
N = 14541
NP = 14544
NLANE = 16
CH = NP // NLANE
NBINS = 256

BB = 512
BN = 2048


def _score_body(q_ref, ent_ref, score_ref, key_ref):
    j = pl.program_id(1)
    q = q_ref[...]
    e = ent_ref[...]
    s = lax.dot_general(q, e, (((1,), (1,)), ((), ())),
                        preferred_element_type=jnp.float32)
    score_ref[...] = s
    x = lax.bitcast_convert_type(s, jnp.int32)
    ud = jnp.where(x < 0, x, ~x & jnp.int32(0x7FFFFFFF))
    col = j * BN + lax.broadcasted_iota(jnp.int32, s.shape, 1)
    key_ref[...] = jnp.where(col >= N, jnp.int32(-1), ud)


def _scores_and_keys(q, ent):
    grid = (B // BB, pl.cdiv(NP, BN))
    return pl.pallas_call(
        _score_body,
        grid=grid,
        in_specs=[
            pl.BlockSpec((BB, 128), lambda i, j: (i, 0)),
            pl.BlockSpec((BN, 128), lambda i, j: (j, 0)),
        ],
        out_specs=[
            pl.BlockSpec((BB, BN), lambda i, j: (i, j)),
            pl.BlockSpec((BB, BN), lambda i, j: (i, j)),
        ],
        out_shape=[
            jax.ShapeDtypeStruct((B, N), jnp.float32),
            jax.ShapeDtypeStruct((B, NP), jnp.int32),
        ],
    )(q, ent)


NROWIL = 1


def _sort_body(nrows, keys_hbm, out_hbm, *scratch):
    nc = 2
    wid = lax.axis_index("s") * nc + lax.axis_index("c")
    rows_per_w = nrows // (nc * 16)
    stride = rows_per_w // NROWIL
    banks = tuple(scratch[6 * i:6 * i + 6] for i in range(NROWIL))
    lane = lax.iota(jnp.int32, NLANE)
    base_addr = lane * CH
    ones = jnp.ones((NLANE,), jnp.int32)
    zeros = jnp.zeros((NLANE,), jnp.int32)

    UZ = 8
    lane15 = jnp.full((NLANE,), 15, jnp.int32)
    is15 = lane == 15

    def do_rows(r, _):
        rows = tuple(wid * rows_per_w + i * stride + r for i in range(NROWIL))
        for (ka, kb, va, vb, hist, tot), row in zip(banks, rows):
            pltpu.sync_copy(keys_hbm.at[row], ka)
        for p in range(4):
            shift = 8 * p
            inbufs = [((ka, va), (kb, vb))[p % 2]
                      for (ka, kb, va, vb, hist, tot) in banks]
            outbufs = [((ka, va), (kb, vb))[(p + 1) % 2]
                       for (ka, kb, va, vb, hist, tot) in banks]
            hists = [bank[4] for bank in banks]
            tots = [bank[5] for bank in banks]

            def hidx_of(k):
                return (((lax.shift_right_logical(k, shift) & 255) << 4)
                        + lane)

            def zero_body(i, _):
                for u in range(UZ):
                    for hist in hists:
                        hist[pl.ds((i * UZ + u) * NLANE, NLANE)] = zeros
                return None

            lax.fori_loop(0, NBINS // UZ, zero_body, None)

            def hist_body(i, carry):
                tn = jnp.minimum(i + 1, CH - 1)
                k_n = [plsc.load_gather(kin, [base_addr + tn])
                       for (kin, vin) in inbufs]
                for hist, hx in zip(hists, carry):
                    plsc.addupdate_scatter(hist, [hx], ones)
                return tuple(hidx_of(k) for k in k_n)

            hx0 = tuple(
                hidx_of(plsc.load_gather(kin, [base_addr]))
                for (kin, vin) in inbufs)
            lax.fori_loop(0, CH, hist_body, hx0)

            def pfx_a(d, carry):
                dn = jnp.minimum(d + 2, NBINS - 1)
                v_nn = [hist[pl.ds(dn * NLANE, NLANE)] for hist in hists]
                for hist, tot, (v_next, incl_cur, v_cur) in zip(
                        hists, tots, carry):
                    hist[pl.ds(d * NLANE, NLANE)] = incl_cur - v_cur
                    plsc.store_scatter(tot, [lane * 0 + d], incl_cur,
                                       mask=is15)
                return tuple(
                    (vn, plsc.cumsum(v_next), v_next)
                    for vn, (v_next, incl_cur, v_cur) in zip(v_nn, carry))

            pfa0 = []
            for hist in hists:
                v0 = hist[pl.ds(0, NLANE)]
                v1 = hist[pl.ds(NLANE, NLANE)]
                pfa0.append((v1, plsc.cumsum(v0), v0))
            lax.fori_loop(0, NBINS, pfx_a, tuple(pfa0))

            def pfx_b(j, carries):
                new = []
                for tot, carry in zip(tots, carries):
                    tv = tot[pl.ds(j * NLANE, NLANE)]
                    tincl = plsc.cumsum(tv)
                    tot[pl.ds(j * NLANE, NLANE)] = tincl - tv + carry
                    new.append(carry + jnp.take(tincl, lane15))
                return tuple(new)

            lax.fori_loop(0, NBINS // NLANE, pfx_b,
                          (jnp.zeros((NLANE,), jnp.int32),) * NROWIL)

            def pfx_c(j, _):
                tvs = [tot[pl.ds(j * NLANE, NLANE)] for tot in tots]
                hs = [[hist[pl.ds((j * NLANE + u) * NLANE, NLANE)]
                       for u in range(NLANE)] for hist in hists]
                for u in range(NLANE):
                    uvec = jnp.full((NLANE,), u, jnp.int32)
                    for hist, tv, h in zip(hists, tvs, hs):
                        hist[pl.ds((j * NLANE + u) * NLANE, NLANE)] = (
                            h[u] + jnp.take(tv, uvec)
                        )
                return None

            lax.fori_loop(0, NBINS // NLANE, pfx_c, None)

            def perm_body(i, carry):
                tn = jnp.minimum(i + 1, CH - 1)
                k_n = [plsc.load_gather(kin, [base_addr + tn])
                       for (kin, vin) in inbufs]
                v_n = ([None] * NROWIL if p == 0 else
                       [plsc.load_gather(vin, [base_addr + tn])
                        for (kin, vin) in inbufs])
                pos = [plsc.load_gather(hist, [hx])
                       for hist, (k_c, hx, v_c) in zip(hists, carry)]
                for hist, (kout, vout), (k_c, hx, v_c), ps in zip(
                        hists, outbufs, carry, pos):
                    plsc.store_scatter(hist, [hx], ps + 1)
                    if p < 3:
                        plsc.store_scatter(kout, [ps], k_c)
                    vv = base_addr + i if p == 0 else v_c
                    plsc.store_scatter(vout, [ps], vv)
                return tuple(
                    (kn, hidx_of(kn), vn) for kn, vn in zip(k_n, v_n))

            pc0 = []
            for (kin, vin) in inbufs:
                k0 = plsc.load_gather(kin, [base_addr])
                v0 = None if p == 0 else plsc.load_gather(vin, [base_addr])
                pc0.append((k0, hidx_of(k0), v0))
            lax.fori_loop(0, CH, perm_body, tuple(pc0))
        for (ka, kb, va, vb, hist, tot), row in zip(banks, rows):
            pltpu.sync_copy(va, out_hbm.at[row])
        return None

    lax.fori_loop(0, stride, do_rows, None)


def _sc_argsort(keys, interpret=False):
    nrows, width = keys.shape
    mesh = plsc.VectorSubcoreMesh(core_axis_name="c", subcore_axis_name="s",
                                  num_cores=2, num_subcores=16)
    f = pl.kernel(
        functools.partial(_sort_body, nrows),
        out_type=jax.ShapeDtypeStruct((nrows, width), jnp.int32),
        mesh=mesh,
        scratch_types=[
            pltpu.VMEM(
                ((width,), (width,), (width,), (width,),
                 (NBINS * NLANE,), (NBINS,))[j], jnp.int32)
            for _ in range(NROWIL) for j in range(6)
        ],
        compiler_params=pltpu.CompilerParams(needs_layout_passes=False),
        interpret=interpret,
    )
    return f(keys)


def kernel(predict_h, predict_r, ent_embeddings, rel_embeddings):
    p_e_h = jnp.take(ent_embeddings, predict_h, axis=0)
    p_e_r = jnp.take(rel_embeddings, predict_r, axis=0)
    q = p_e_h * p_e_r
    p_score, keys = _scores_and_keys(q, ent_embeddings)
    ranking = _sc_argsort(keys)[:, :N]
    return p_score, ranking


# speedup vs baseline: 1.5344x; 1.5344x over previous
import functools

import jax
import jax.numpy as jnp
from jax import lax
from jax.experimental import pallas as pl
from jax.experimental.pallas import tpu as pltpu
from jax.experimental.pallas import tpu_sc as plsc

B = 4096
N = 14541
NP = 14544
NLANE = 16
CH = NP // NLANE
NBINS = 256

BB = 512
BN = 2048


def _score_body(q_ref, ent_ref, score_ref, key_ref):
    j = pl.program_id(1)
    q = q_ref[...]
    e = ent_ref[...]
    s = lax.dot_general(q, e, (((1,), (1,)), ((), ())),
                        preferred_element_type=jnp.float32)
    score_ref[...] = s
    x = lax.bitcast_convert_type(s, jnp.int32)
    ud = jnp.where(x < 0, x, ~x & jnp.int32(0x7FFFFFFF))
    col = j * BN + lax.broadcasted_iota(jnp.int32, s.shape, 1)
    key_ref[...] = jnp.where(col >= N, jnp.int32(-1), ud)


def _scores_and_keys(q, ent):
    grid = (B // BB, pl.cdiv(NP, BN))
    return pl.pallas_call(
        _score_body,
        grid=grid,
        in_specs=[
            pl.BlockSpec((BB, 128), lambda i, j: (i, 0)),
            pl.BlockSpec((BN, 128), lambda i, j: (j, 0)),
        ],
        out_specs=[
            pl.BlockSpec((BB, BN), lambda i, j: (i, j)),
            pl.BlockSpec((BB, BN), lambda i, j: (i, j)),
        ],
        out_shape=[
            jax.ShapeDtypeStruct((B, N), jnp.float32),
            jax.ShapeDtypeStruct((B, NP), jnp.int32),
        ],
    )(q, ent)


NROWIL = 2


def _sort_body(nrows, keys_hbm, out_hbm, *scratch):
    nc = 2
    wid = lax.axis_index("s") * nc + lax.axis_index("c")
    rows_per_w = nrows // (nc * 16)
    stride = rows_per_w // NROWIL
    banks = tuple(scratch[6 * i:6 * i + 6] for i in range(NROWIL))
    lane = lax.iota(jnp.int32, NLANE)
    base_addr = lane * CH
    ones = jnp.ones((NLANE,), jnp.int32)
    zeros = jnp.zeros((NLANE,), jnp.int32)

    UZ = 8
    lane15 = jnp.full((NLANE,), 15, jnp.int32)
    is15 = lane == 15

    def do_rows(r, _):
        rows = tuple(wid * rows_per_w + i * stride + r for i in range(NROWIL))
        for (ka, kb, va, vb, hist, tot), row in zip(banks, rows):
            pltpu.sync_copy(keys_hbm.at[row], ka)
        cfg = [
            (0,  0, None, 1, 3),
            (8,  1, 3,    2, None),
            (16, 2, None, 3, None),
            (24, 3, None, 2, None),
        ]
        for p, (shift, si, vi, oi, o2i) in enumerate(cfg):
            inbufs = [bank[si] for bank in banks]
            vbufs = [None if vi is None else bank[vi] for bank in banks]
            outbufs = [(bank[oi], None if o2i is None else bank[o2i])
                       for bank in banks]
            hists = [bank[4] for bank in banks]
            tots = [bank[5] for bank in banks]

            def hidx_of(k):
                return (((lax.shift_right_logical(k, shift) & 255) << 4)
                        + lane)

            def zero_body(i, _):
                for u in range(UZ):
                    for hist in hists:
                        hist[pl.ds((i * UZ + u) * NLANE, NLANE)] = zeros
                return None

            lax.fori_loop(0, NBINS // UZ, zero_body, None)

            def hist_body(i, carry):
                tn = jnp.minimum(i + 1, CH - 1)
                k_n = [plsc.load_gather(kin, [base_addr + tn])
                       for kin in inbufs]
                for hist, hx in zip(hists, carry):
                    plsc.addupdate_scatter(hist, [hx], ones)
                return tuple(hidx_of(k) for k in k_n)

            hx0 = tuple(
                hidx_of(plsc.load_gather(kin, [base_addr]))
                for kin in inbufs)
            lax.fori_loop(0, CH, hist_body, hx0)

            def pfx_a(d, carry):
                dn = jnp.minimum(d + 2, NBINS - 1)
                v_nn = [hist[pl.ds(dn * NLANE, NLANE)] for hist in hists]
                for hist, tot, (v_next, incl_cur, v_cur) in zip(
                        hists, tots, carry):
                    hist[pl.ds(d * NLANE, NLANE)] = incl_cur - v_cur
                    plsc.store_scatter(tot, [lane * 0 + d], incl_cur,
                                       mask=is15)
                return tuple(
                    (vn, plsc.cumsum(v_next), v_next)
                    for vn, (v_next, incl_cur, v_cur) in zip(v_nn, carry))

            pfa0 = []
            for hist in hists:
                v0 = hist[pl.ds(0, NLANE)]
                v1 = hist[pl.ds(NLANE, NLANE)]
                pfa0.append((v1, plsc.cumsum(v0), v0))
            lax.fori_loop(0, NBINS, pfx_a, tuple(pfa0))

            def pfx_b(j, carries):
                new = []
                for tot, carry in zip(tots, carries):
                    tv = tot[pl.ds(j * NLANE, NLANE)]
                    tincl = plsc.cumsum(tv)
                    tot[pl.ds(j * NLANE, NLANE)] = tincl - tv + carry
                    new.append(carry + jnp.take(tincl, lane15))
                return tuple(new)

            lax.fori_loop(0, NBINS // NLANE, pfx_b,
                          (jnp.zeros((NLANE,), jnp.int32),) * NROWIL)

            def pfx_c(j, _):
                tvs = [tot[pl.ds(j * NLANE, NLANE)] for tot in tots]
                hs = [[hist[pl.ds((j * NLANE + u) * NLANE, NLANE)]
                       for u in range(NLANE)] for hist in hists]
                for u in range(NLANE):
                    uvec = jnp.full((NLANE,), u, jnp.int32)
                    for hist, tv, h in zip(hists, tvs, hs):
                        hist[pl.ds((j * NLANE + u) * NLANE, NLANE)] = (
                            h[u] + jnp.take(tv, uvec)
                        )
                return None

            lax.fori_loop(0, NBINS // NLANE, pfx_c, None)

            def perm_body(i, carry):
                tn = jnp.minimum(i + 1, CH - 1)
                k_n = [plsc.load_gather(kin, [base_addr + tn])
                       for kin in inbufs]
                v_n = [None if vin is None else
                       plsc.load_gather(vin, [base_addr + tn])
                       for vin in vbufs]
                pos = [plsc.load_gather(hist, [hx])
                       for hist, (k_c, hx, v_c) in zip(hists, carry)]
                for hist, (out1, out2), (k_c, hx, v_c), ps in zip(
                        hists, outbufs, carry, pos):
                    plsc.store_scatter(hist, [hx], ps + 1)
                    if p == 0:
                        plsc.store_scatter(out1, [ps], k_c)
                        plsc.store_scatter(out2, [ps], base_addr + i)
                    elif p == 1:
                        w = (k_c & jnp.int32(-65536)) | v_c
                        plsc.store_scatter(out1, [ps], w)
                    elif p == 2:
                        plsc.store_scatter(out1, [ps], k_c)
                    else:
                        plsc.store_scatter(out1, [ps],
                                           k_c & jnp.int32(0x3FFF))
                return tuple(
                    (kn, hidx_of(kn), vn) for kn, vn in zip(k_n, v_n))

            pc0 = []
            for kin, vin in zip(inbufs, vbufs):
                k0 = plsc.load_gather(kin, [base_addr])
                v0 = (None if vin is None
                      else plsc.load_gather(vin, [base_addr]))
                pc0.append((k0, hidx_of(k0), v0))
            lax.fori_loop(0, CH, perm_body, tuple(pc0))
        for (ka, kb, va, vb, hist, tot), row in zip(banks, rows):
            pltpu.sync_copy(va, out_hbm.at[row])
        return None

    lax.fori_loop(0, stride, do_rows, None)


def _sc_argsort(keys, interpret=False):
    nrows, width = keys.shape
    mesh = plsc.VectorSubcoreMesh(core_axis_name="c", subcore_axis_name="s",
                                  num_cores=2, num_subcores=16)
    f = pl.kernel(
        functools.partial(_sort_body, nrows),
        out_type=jax.ShapeDtypeStruct((nrows, width), jnp.int32),
        mesh=mesh,
        scratch_types=[
            pltpu.VMEM(
                ((width,), (width,), (width,), (width,),
                 (NBINS * NLANE,), (NBINS,))[j], jnp.int32)
            for _ in range(NROWIL) for j in range(6)
        ],
        compiler_params=pltpu.CompilerParams(needs_layout_passes=False),
        interpret=interpret,
    )
    return f(keys)


def kernel(predict_h, predict_r, ent_embeddings, rel_embeddings):
    p_e_h = jnp.take(ent_embeddings, predict_h, axis=0)
    p_e_r = jnp.take(rel_embeddings, predict_r, axis=0)
    q = p_e_h * p_e_r
    p_score, keys = _scores_and_keys(q, ent_embeddings)
    ranking = _sc_argsort(keys)[:, :N]
    return p_score, ranking
